# baseline (speedup 1.0000x reference)
import functools

import jax
import jax.numpy as jnp
from jax import lax
from jax.experimental import pallas as pl
from jax.experimental.pallas import tpu as pltpu
from jax.experimental.pallas import tpu_sc as plsc

N_NODES = 10000
N_EDGES = 320000
C_IN = 128
C_OUT = 128

D = 144
K = 124
NW = 32
CPT = 84
BLK = 8
IDXR = CPT + 4
E_PAD = NW * CPT * K
N_PAD = 10016
RPT = N_PAD // 16
TRASH = N_NODES


def _proj_body(x_ref, w_ref, a_ref, bm_ref):
    x = x_ref[...]
    w = w_ref[...]
    w1 = w[:, :C_IN]
    w2 = w[:, C_IN:]
    dn = (((0,), (1,)), ((), ()))
    a_ref[...] = lax.dot_general(x, w1 - w2, dn, preferred_element_type=jnp.float32)
    bm = lax.dot_general(x, w2, dn, preferred_element_type=jnp.float32)
    ones = jnp.ones((N_PAD, 1), jnp.float32)
    zeros = jnp.zeros((N_PAD, D - C_OUT - 1), jnp.float32)
    bm_ref[...] = jnp.concatenate([bm, ones, zeros], axis=1)


_proj = pl.pallas_call(
    _proj_body,
    out_shape=[
        jax.ShapeDtypeStruct((N_PAD, C_OUT), jnp.float32),
        jax.ShapeDtypeStruct((N_PAD, D), jnp.float32),
    ],
)


def _sc_body(table, g_hbm, r_hbm, z_hbm, out, g_v, r_v, rows_v, acc, sem_g, sem_s):
    cid = lax.axis_index("c")
    sid = lax.axis_index("s")
    row0 = sid * RPT
    pltpu.sync_copy(z_hbm, acc.at[pl.ds(row0, RPT)])
    wid = sid * 2 + cid
    pltpu.sync_copy(g_hbm.at[wid, pl.ds(0, BLK)], g_v.at[0])
    pltpu.sync_copy(r_hbm.at[wid, pl.ds(0, BLK)], r_v.at[0])
    plsc.subcore_barrier()

    def g_row(j):
        return g_v.at[(j // BLK) % 2, j % BLK]

    def r_row(j):
        return r_v.at[(j // BLK) % 2, j % BLK]

    def issue_g(j):
        pltpu.async_copy(table.at[g_row(j)], rows_v.at[j % 2], sem_g)

    def wait_g(j):
        pltpu.make_async_copy(table.at[g_row(j)], rows_v.at[j % 2], sem_g).wait()

    def issue_s(j):
        pltpu.async_copy(rows_v.at[j % 2], acc.at[r_row(j)], sem_s, add=True)

    def wait_s(j):
        pltpu.make_async_copy(rows_v.at[j % 2], acc.at[r_row(j)], sem_s).wait()

    issue_g(0)
    wait_g(0)
    issue_g(1)
    issue_s(0)

    def body(j, carry):
        @pl.when(lax.rem(j, BLK) == BLK // 2)
        def _():
            blk = lax.div(j, BLK) + 1
            p = lax.rem(blk, 2)
            pltpu.sync_copy(g_hbm.at[wid, pl.ds(blk * BLK, BLK)], g_v.at[p])
            pltpu.sync_copy(r_hbm.at[wid, pl.ds(blk * BLK, BLK)], r_v.at[p])
        wait_g(j)
        wait_s(j - 1)
        issue_g(j + 1)
        issue_s(j)
        return carry

    lax.fori_loop(1, CPT - 1, body, 0)
    wait_g(CPT - 1)
    wait_s(CPT - 2)
    issue_s(CPT - 1)
    wait_s(CPT - 1)
    plsc.subcore_barrier()
    pltpu.sync_copy(acc.at[pl.ds(row0, RPT)], out.at[cid, pl.ds(row0, RPT)])


@functools.cache
def _sc_scatter():
    return pl.kernel(
        _sc_body,
        mesh=plsc.VectorSubcoreMesh(core_axis_name="c", subcore_axis_name="s"),
        compiler_params=pltpu.CompilerParams(use_tc_tiling_on_sc=False),
        out_type=jax.ShapeDtypeStruct((2, N_PAD, D), jnp.float32),
        scratch_types=[
            pltpu.VMEM((2, BLK, K), jnp.int32),
            pltpu.VMEM((2, BLK, K), jnp.int32),
            pltpu.VMEM((2, K, D), jnp.float32),
            pltpu.VMEM_SHARED((N_PAD, D), jnp.float32),
            pltpu.SemaphoreType.DMA,
            pltpu.SemaphoreType.DMA,
        ],
    )


def _comb_body(a_ref, s_ref, b_ref, o_ref):
    s = s_ref[0] + s_ref[1]
    sums = s[:, :C_OUT]
    cnt = s[:, C_OUT:C_OUT + 1]
    val = a_ref[...] + b_ref[...] + sums / jnp.maximum(cnt, 1.0)
    val = jnp.where(cnt > 0, val, 0.0)
    o_ref[...] = jnp.where(val > 0, val, 0.3 * val)


_comb = pl.pallas_call(
    _comb_body,
    out_shape=jax.ShapeDtypeStruct((N_PAD, C_OUT), jnp.float32),
)


def kernel(in_features, reduce_index, gather_index, W, b):
    x = in_features[0]
    x_pad = jnp.pad(x, ((0, 0), (0, N_PAD - N_NODES)))
    pad = jnp.full((E_PAD - N_EDGES,), TRASH, jnp.int32)
    blkpad = jnp.full((NW, IDXR - CPT, K), TRASH, jnp.int32)
    g_idx = jnp.concatenate([gather_index, pad]).reshape(NW, CPT, K)
    g_idx = jnp.concatenate([g_idx, blkpad], axis=1)
    r_idx = jnp.concatenate([reduce_index, pad]).reshape(NW, CPT, K)
    r_idx = jnp.concatenate([r_idx, blkpad], axis=1)
    zeros = jnp.zeros((RPT, D), jnp.float32)

    a_t, table = _proj(x_pad, W)
    partials = _sc_scatter()(table, g_idx, r_idx, zeros)
    out_t = _comb(a_t, partials, b.reshape(1, C_OUT))
    return jnp.transpose(out_t[:N_NODES])[None]


# speedup vs baseline: 1.6389x; 1.6389x over previous
import functools

import jax
import jax.numpy as jnp
from jax import lax
from jax.experimental import pallas as pl
from jax.experimental.pallas import tpu as pltpu
from jax.experimental.pallas import tpu_sc as plsc

N_NODES = 10000
N_EDGES = 320000
C_IN = 128
C_OUT = 128

D = 160
K = 128
NW = 32
CPT = 80
E_PAD = NW * CPT * K
N_PAD = 10016
RPT = N_PAD // 16
TRASH = N_NODES


def _proj_body(x_ref, w_ref, a_ref, bm_ref):
    x = x_ref[...]
    w = w_ref[...]
    w1 = w[:, :C_IN]
    w2 = w[:, C_IN:]
    dn = (((0,), (1,)), ((), ()))
    a_ref[...] = lax.dot_general(x, w1 - w2, dn, preferred_element_type=jnp.float32)
    bm = lax.dot_general(x, w2, dn, preferred_element_type=jnp.float32)
    ones = jnp.ones((N_PAD, 1), jnp.float32)
    zeros = jnp.zeros((N_PAD, D - C_OUT - 1), jnp.float32)
    bm_ref[...] = jnp.concatenate([bm, ones, zeros], axis=1).astype(jnp.bfloat16)


_proj = pl.pallas_call(
    _proj_body,
    out_shape=[
        jax.ShapeDtypeStruct((N_PAD, C_OUT), jnp.float32),
        jax.ShapeDtypeStruct((N_PAD, D), jnp.bfloat16),
    ],
)


def _sc_body(table, g_hbm, r_hbm, z_hbm, out, g_v, r_v, rows_v, acc, sem):
    cid = lax.axis_index("c")
    sid = lax.axis_index("s")
    row0 = sid * RPT
    pltpu.sync_copy(z_hbm, acc.at[pl.ds(row0, RPT)])
    wid = sid * 2 + cid
    pltpu.sync_copy(g_hbm.at[wid], g_v)
    pltpu.sync_copy(r_hbm.at[wid], r_v)
    plsc.subcore_barrier()

    def body(j, carry):
        pltpu.async_copy(table.at[g_v.at[j]], rows_v, sem).wait()
        pltpu.sync_copy(rows_v, acc.at[r_v.at[j]], add=True)
        return carry

    lax.fori_loop(0, CPT, body, 0)
    plsc.subcore_barrier()
    pltpu.sync_copy(acc.at[pl.ds(row0, RPT)], out.at[cid, pl.ds(row0, RPT)])


@functools.cache
def _sc_scatter():
    return pl.kernel(
        _sc_body,
        mesh=plsc.VectorSubcoreMesh(core_axis_name="c", subcore_axis_name="s"),
        compiler_params=pltpu.CompilerParams(use_tc_tiling_on_sc=False),
        out_type=jax.ShapeDtypeStruct((2, N_PAD, D), jnp.bfloat16),
        scratch_types=[
            pltpu.VMEM((CPT, K), jnp.int32),
            pltpu.VMEM((CPT, K), jnp.int32),
            pltpu.VMEM((K, D), jnp.bfloat16),
            pltpu.VMEM_SHARED((N_PAD, D), jnp.bfloat16),
            pltpu.SemaphoreType.DMA,
        ],
    )


def _comb_body(a_ref, s_ref, b_ref, o_ref):
    s = s_ref[0].astype(jnp.float32) + s_ref[1].astype(jnp.float32)
    sums = s[:, :C_OUT]
    cnt = s[:, C_OUT:C_OUT + 1]
    val = a_ref[...] + b_ref[...] + sums / jnp.maximum(cnt, 1.0)
    val = jnp.where(cnt > 0, val, 0.0)
    o_ref[...] = jnp.where(val > 0, val, 0.3 * val)


_comb = pl.pallas_call(
    _comb_body,
    out_shape=jax.ShapeDtypeStruct((N_PAD, C_OUT), jnp.float32),
)


def kernel(in_features, reduce_index, gather_index, W, b):
    x = in_features[0]
    x_pad = jnp.pad(x, ((0, 0), (0, N_PAD - N_NODES)))
    pad = jnp.full((E_PAD - N_EDGES,), TRASH, jnp.int32)
    g_idx = jnp.concatenate([gather_index, pad]).reshape(NW, CPT, K)
    r_idx = jnp.concatenate([reduce_index, pad]).reshape(NW, CPT, K)
    zeros = jnp.zeros((RPT, D), jnp.bfloat16)

    a_t, table = _proj(x_pad, W)
    partials = _sc_scatter()(table, g_idx, r_idx, zeros)
    out_t = _comb(a_t, partials, b.reshape(1, C_OUT))
    return jnp.transpose(out_t[:N_NODES])[None]
